# Initial kernel scaffold; baseline (speedup 1.0000x reference)
import jax
import jax.numpy as jnp
from jax.experimental import pallas as pl


def kernel(x, Wr, Wg, Wu, Wd, gamma, beta):
    raise NotImplementedError("write your pallas kernel here")


# speedup vs baseline: 3.0625x; 3.0625x over previous
import functools
import math

import jax
import jax.numpy as jnp
from jax.experimental import pallas as pl
from jax.experimental.pallas import tpu as pltpu


def _routing_kernel(x_ref, wr_ref, sel_ref, gates_ref, lb_ref, *, capacity):
    x = x_ref[...]
    wr = wr_ref[...]
    N = x.shape[0]
    E = wr.shape[0]
    logits = jax.lax.dot_general(x, wr, (((1,), (1,)), ((), ())),
                                 preferred_element_type=jnp.float32)
    m = jnp.max(logits, axis=-1, keepdims=True)
    ex = jnp.exp(logits - m)
    probs = ex / jnp.sum(ex, axis=-1, keepdims=True)
    top1_val = jnp.max(probs, axis=-1, keepdims=True)
    lane = jax.lax.broadcasted_iota(jnp.int32, (N, E), 1)
    is_max = probs == top1_val
    top1_idx = jnp.min(jnp.where(is_max, lane, E), axis=-1, keepdims=True)
    onehot = (lane == top1_idx).astype(jnp.float32)

    row_i = jax.lax.broadcasted_iota(jnp.int32, (N, N), 0)
    col_i = jax.lax.broadcasted_iota(jnp.int32, (N, N), 1)
    tril = (row_i >= col_i).astype(jnp.float32)
    ranks_incl = jnp.dot(tril, onehot, preferred_element_type=jnp.float32)
    pos = ranks_incl.astype(jnp.int32) - 1

    keep = (onehot > 0.5) & (pos < capacity)
    sel_ref[...] = jnp.where(keep, pos, -1)
    gates_ref[...] = jnp.where(keep, top1_val, 0.0)

    counts = jnp.sum(onehot, axis=0, keepdims=True) / N
    pmean = jnp.sum(probs, axis=0, keepdims=True) / N
    lb_ref[...] = E * jnp.sum(counts * pmean, axis=-1, keepdims=True)


def _moe_kernel(x_ref, sel_ref, gates_ref, wg_ref, wu_ref, wd_ref,
                gamma_ref, beta_ref, out_ref,
                xe_ref, s_ref, yacc_ref, acc_ref, *, nf, n_experts, capacity):
    e = pl.program_id(0)
    f = pl.program_id(1)
    N, D = x_ref.shape

    @pl.when(f == 0)
    def _gather():
        sel_row = sel_ref[0, 0, :]
        slot = jax.lax.broadcasted_iota(jnp.int32, (capacity, N), 0)
        s = (sel_row[None, :] == slot).astype(jnp.float32)
        s_ref[...] = s
        xe_ref[...] = jax.lax.dot_general(
            s, x_ref[...], (((1,), (0,)), ((), ())),
            preferred_element_type=jnp.float32)

    xe = xe_ref[...]
    g = jnp.dot(xe, wg_ref[0], preferred_element_type=jnp.float32)
    u = jnp.dot(xe, wu_ref[0], preferred_element_type=jnp.float32)
    h = g * jax.lax.logistic(g) * u
    dy = jnp.dot(h, wd_ref[0], preferred_element_type=jnp.float32)

    @pl.when(f == 0)
    def _init_y():
        yacc_ref[...] = dy

    @pl.when(f != 0)
    def _acc_y():
        yacc_ref[...] += dy

    @pl.when(f == nf - 1)
    def _combine():
        contrib = jax.lax.dot_general(
            s_ref[...], yacc_ref[...], (((0,), (0,)), ((), ())),
            preferred_element_type=jnp.float32)
        contrib = contrib * gates_ref[0, 0, :][:, None]

        @pl.when(e == 0)
        def _():
            acc_ref[...] = contrib

        @pl.when(e != 0)
        def _():
            acc_ref[...] += contrib

        @pl.when(e == n_experts - 1)
        def _layernorm():
            y = acc_ref[...] + x_ref[...]
            mu = jnp.mean(y, axis=-1, keepdims=True)
            yc = y - mu
            var = jnp.mean(yc * yc, axis=-1, keepdims=True)
            inv = jax.lax.rsqrt(var + 1e-5)
            out_ref[...] = yc * inv * gamma_ref[0] + beta_ref[0]


@jax.jit
def kernel(x, Wr, Wg, Wu, Wd, gamma, beta):
    B, T, D = x.shape
    N = B * T
    E, _, F = Wg.shape
    capacity = math.ceil(N / E * 1.25)
    C = capacity
    x_flat = x.reshape(N, D)

    sel, gates, lb = pl.pallas_call(
        functools.partial(_routing_kernel, capacity=C),
        out_shape=[
            jax.ShapeDtypeStruct((N, E), jnp.int32),
            jax.ShapeDtypeStruct((N, E), jnp.float32),
            jax.ShapeDtypeStruct((1, 1), jnp.float32),
        ],
    )(x_flat, Wr)

    sel_t = sel.T.reshape(E, 1, N)
    gates_t = gates.T.reshape(E, 1, N)

    FB = 256
    NF = F // FB
    out = pl.pallas_call(
        functools.partial(_moe_kernel, nf=NF, n_experts=E, capacity=C),
        grid=(E, NF),
        in_specs=[
            pl.BlockSpec((N, D), lambda e, f: (0, 0)),
            pl.BlockSpec((1, 1, N), lambda e, f: (e, 0, 0)),
            pl.BlockSpec((1, 1, N), lambda e, f: (e, 0, 0)),
            pl.BlockSpec((1, D, FB), lambda e, f: (e, 0, f)),
            pl.BlockSpec((1, D, FB), lambda e, f: (e, 0, f)),
            pl.BlockSpec((1, FB, D), lambda e, f: (e, f, 0)),
            pl.BlockSpec((1, D), lambda e, f: (0, 0)),
            pl.BlockSpec((1, D), lambda e, f: (0, 0)),
        ],
        out_specs=pl.BlockSpec((N, D), lambda e, f: (0, 0)),
        out_shape=jax.ShapeDtypeStruct((N, D), jnp.float32),
        scratch_shapes=[
            pltpu.VMEM((C, D), jnp.float32),
            pltpu.VMEM((C, N), jnp.float32),
            pltpu.VMEM((C, D), jnp.float32),
            pltpu.VMEM((N, D), jnp.float32),
        ],
        compiler_params=pltpu.CompilerParams(
            dimension_semantics=("arbitrary", "arbitrary"),
        ),
    )(x_flat, sel_t, gates_t, Wg, Wu, Wd,
      gamma.reshape(1, D), beta.reshape(1, D))

    return out.reshape(B, T, D), lb[0, 0]


# speedup vs baseline: 3.0815x; 1.0062x over previous
import functools
import math

import jax
import jax.numpy as jnp
from jax.experimental import pallas as pl
from jax.experimental.pallas import tpu as pltpu


def _routing_kernel(x_ref, wr_ref, sel_ref, gates_ref, lb_ref, *, capacity):
    x = x_ref[...]
    wr = wr_ref[...]
    N = x.shape[0]
    E = wr.shape[0]
    logits = jax.lax.dot_general(x, wr, (((1,), (1,)), ((), ())),
                                 preferred_element_type=jnp.float32)
    m = jnp.max(logits, axis=-1, keepdims=True)
    ex = jnp.exp(logits - m)
    probs = ex / jnp.sum(ex, axis=-1, keepdims=True)
    top1_val = jnp.max(probs, axis=-1, keepdims=True)
    lane = jax.lax.broadcasted_iota(jnp.int32, (N, E), 1)
    is_max = probs == top1_val
    top1_idx = jnp.min(jnp.where(is_max, lane, E), axis=-1, keepdims=True)
    onehot = (lane == top1_idx).astype(jnp.float32)

    row_i = jax.lax.broadcasted_iota(jnp.int32, (N, N), 0)
    col_i = jax.lax.broadcasted_iota(jnp.int32, (N, N), 1)
    tril = (row_i >= col_i).astype(jnp.float32)
    ranks_incl = jnp.dot(tril, onehot, preferred_element_type=jnp.float32)
    pos = ranks_incl.astype(jnp.int32) - 1

    keep = (onehot > 0.5) & (pos < capacity)
    sel_ref[...] = jnp.where(keep, pos, -1)
    gates_ref[...] = jnp.where(keep, top1_val, 0.0)

    counts = jnp.sum(onehot, axis=0, keepdims=True) / N
    pmean = jnp.sum(probs, axis=0, keepdims=True) / N
    lb_ref[...] = E * jnp.sum(counts * pmean, axis=-1, keepdims=True)


def _moe_kernel(x_ref, sel_ref, gates_ref, wg_ref, wu_ref, wd_ref,
                gamma_ref, beta_ref, out_ref,
                xbf_ref, xe_ref, s_ref, yacc_ref, acc_ref,
                *, nf, n_experts, capacity):
    e = pl.program_id(0)
    f = pl.program_id(1)
    N, D = x_ref.shape

    @pl.when((e == 0) & (f == 0))
    def _cast_x():
        xbf_ref[...] = x_ref[...].astype(jnp.bfloat16)

    @pl.when(f == 0)
    def _gather():
        sel_row = sel_ref[0, 0, :]
        slot = jax.lax.broadcasted_iota(jnp.int32, (capacity, N), 0)
        s = (sel_row[None, :] == slot).astype(jnp.bfloat16)
        s_ref[...] = s
        xe_ref[...] = jax.lax.dot_general(
            s, xbf_ref[...], (((1,), (0,)), ((), ())),
            preferred_element_type=jnp.float32).astype(jnp.bfloat16)

    xe = xe_ref[...]
    wg = wg_ref[0].astype(jnp.bfloat16)
    wu = wu_ref[0].astype(jnp.bfloat16)
    wd = wd_ref[0].astype(jnp.bfloat16)
    g = jnp.dot(xe, wg, preferred_element_type=jnp.float32)
    u = jnp.dot(xe, wu, preferred_element_type=jnp.float32)
    h = (g * jax.lax.logistic(g) * u).astype(jnp.bfloat16)
    dy = jnp.dot(h, wd, preferred_element_type=jnp.float32)

    @pl.when(f == 0)
    def _init_y():
        yacc_ref[...] = dy

    @pl.when(f != 0)
    def _acc_y():
        yacc_ref[...] += dy

    @pl.when(f == nf - 1)
    def _combine():
        contrib = jax.lax.dot_general(
            s_ref[...], yacc_ref[...].astype(jnp.bfloat16),
            (((0,), (0,)), ((), ())),
            preferred_element_type=jnp.float32)
        contrib = contrib * gates_ref[0, 0, :][:, None]

        @pl.when(e == 0)
        def _():
            acc_ref[...] = contrib

        @pl.when(e != 0)
        def _():
            acc_ref[...] += contrib

        @pl.when(e == n_experts - 1)
        def _layernorm():
            y = acc_ref[...] + x_ref[...]
            mu = jnp.mean(y, axis=-1, keepdims=True)
            yc = y - mu
            var = jnp.mean(yc * yc, axis=-1, keepdims=True)
            inv = jax.lax.rsqrt(var + 1e-5)
            out_ref[...] = yc * inv * gamma_ref[0] + beta_ref[0]


@jax.jit
def kernel(x, Wr, Wg, Wu, Wd, gamma, beta):
    B, T, D = x.shape
    N = B * T
    E, _, F = Wg.shape
    capacity = math.ceil(N / E * 1.25)
    C = capacity
    x_flat = x.reshape(N, D)

    sel, gates, lb = pl.pallas_call(
        functools.partial(_routing_kernel, capacity=C),
        out_shape=[
            jax.ShapeDtypeStruct((N, E), jnp.int32),
            jax.ShapeDtypeStruct((N, E), jnp.float32),
            jax.ShapeDtypeStruct((1, 1), jnp.float32),
        ],
    )(x_flat, Wr)

    sel_t = sel.T.reshape(E, 1, N)
    gates_t = gates.T.reshape(E, 1, N)

    FB = 256
    NF = F // FB
    out = pl.pallas_call(
        functools.partial(_moe_kernel, nf=NF, n_experts=E, capacity=C),
        grid=(E, NF),
        in_specs=[
            pl.BlockSpec((N, D), lambda e, f: (0, 0)),
            pl.BlockSpec((1, 1, N), lambda e, f: (e, 0, 0)),
            pl.BlockSpec((1, 1, N), lambda e, f: (e, 0, 0)),
            pl.BlockSpec((1, D, FB), lambda e, f: (e, 0, f)),
            pl.BlockSpec((1, D, FB), lambda e, f: (e, 0, f)),
            pl.BlockSpec((1, FB, D), lambda e, f: (e, f, 0)),
            pl.BlockSpec((1, D), lambda e, f: (0, 0)),
            pl.BlockSpec((1, D), lambda e, f: (0, 0)),
        ],
        out_specs=pl.BlockSpec((N, D), lambda e, f: (0, 0)),
        out_shape=jax.ShapeDtypeStruct((N, D), jnp.float32),
        scratch_shapes=[
            pltpu.VMEM((N, D), jnp.bfloat16),
            pltpu.VMEM((C, D), jnp.bfloat16),
            pltpu.VMEM((C, N), jnp.bfloat16),
            pltpu.VMEM((C, D), jnp.float32),
            pltpu.VMEM((N, D), jnp.float32),
        ],
        compiler_params=pltpu.CompilerParams(
            dimension_semantics=("arbitrary", "arbitrary"),
        ),
    )(x_flat, sel_t, gates_t, Wg, Wu, Wd,
      gamma.reshape(1, D), beta.reshape(1, D))

    return out.reshape(B, T, D), lb[0, 0]


# speedup vs baseline: 3.8005x; 1.2333x over previous
import functools
import math

import jax
import jax.numpy as jnp
from jax.experimental import pallas as pl
from jax.experimental.pallas import tpu as pltpu


def _routing_kernel(x_ref, wr_ref, sel_ref, gates_ref, lb_ref, *, capacity):
    x = x_ref[...]
    wr = wr_ref[...]
    N = x.shape[0]
    E = wr.shape[0]
    logits = jax.lax.dot_general(x, wr, (((1,), (1,)), ((), ())),
                                 preferred_element_type=jnp.float32)
    m = jnp.max(logits, axis=-1, keepdims=True)
    ex = jnp.exp(logits - m)
    probs = ex / jnp.sum(ex, axis=-1, keepdims=True)
    top1_val = jnp.max(probs, axis=-1, keepdims=True)
    lane = jax.lax.broadcasted_iota(jnp.int32, (N, E), 1)
    is_max = probs == top1_val
    top1_idx = jnp.min(jnp.where(is_max, lane, E), axis=-1, keepdims=True)
    onehot = (lane == top1_idx).astype(jnp.float32)

    row_i = jax.lax.broadcasted_iota(jnp.int32, (N, N), 0)
    col_i = jax.lax.broadcasted_iota(jnp.int32, (N, N), 1)
    tril = (row_i >= col_i).astype(jnp.float32)
    ranks_incl = jnp.dot(tril, onehot, preferred_element_type=jnp.float32)
    pos = ranks_incl.astype(jnp.int32) - 1

    keep = (onehot > 0.5) & (pos < capacity)
    sel_ref[...] = jnp.where(keep, pos, -1)
    gates_ref[...] = jnp.where(keep, top1_val, 0.0)

    counts = jnp.sum(onehot, axis=0, keepdims=True) / N
    pmean = jnp.sum(probs, axis=0, keepdims=True) / N
    lb_ref[...] = E * jnp.sum(counts * pmean, axis=-1, keepdims=True)


def _moe_kernel(x_ref, sel_ref, gates_ref, wg_ref, wu_ref, wd_ref,
                gamma_ref, beta_ref, out_ref,
                xe_ref, s_ref, yacc_ref, acc_ref,
                *, nf, n_experts, capacity):
    e = pl.program_id(0)
    f = pl.program_id(1)
    N, D = x_ref.shape

    @pl.when(f == 0)
    def _gather():
        sel_row = sel_ref[0, 0, :]
        slot = jax.lax.broadcasted_iota(jnp.int32, (capacity, N), 0)
        s = (sel_row[None, :] == slot).astype(jnp.float32)
        s_ref[...] = s.astype(jnp.bfloat16)
        xe_ref[...] = jax.lax.dot_general(
            s, x_ref[...], (((1,), (0,)), ((), ())),
            preferred_element_type=jnp.float32).astype(jnp.bfloat16)

    xe = xe_ref[...]
    wg = wg_ref[0].astype(jnp.bfloat16)
    wu = wu_ref[0].astype(jnp.bfloat16)
    wd = wd_ref[0].astype(jnp.bfloat16)
    g = jnp.dot(xe, wg, preferred_element_type=jnp.float32)
    u = jnp.dot(xe, wu, preferred_element_type=jnp.float32)
    h = (g * jax.lax.logistic(g) * u).astype(jnp.bfloat16)
    dy = jnp.dot(h, wd, preferred_element_type=jnp.float32)

    @pl.when(f == 0)
    def _init_y():
        yacc_ref[...] = dy

    @pl.when(f != 0)
    def _acc_y():
        yacc_ref[...] += dy

    @pl.when(f == nf - 1)
    def _combine():
        contrib = jax.lax.dot_general(
            s_ref[...], yacc_ref[...].astype(jnp.bfloat16),
            (((0,), (0,)), ((), ())),
            preferred_element_type=jnp.float32)
        contrib = contrib * gates_ref[0, 0, :][:, None]

        @pl.when(e == 0)
        def _():
            acc_ref[...] = contrib

        @pl.when(e != 0)
        def _():
            acc_ref[...] += contrib

        @pl.when(e == n_experts - 1)
        def _layernorm():
            y = acc_ref[...] + x_ref[...]
            mu = jnp.mean(y, axis=-1, keepdims=True)
            yc = y - mu
            var = jnp.mean(yc * yc, axis=-1, keepdims=True)
            inv = jax.lax.rsqrt(var + 1e-5)
            out_ref[...] = yc * inv * gamma_ref[0] + beta_ref[0]


@jax.jit
def kernel(x, Wr, Wg, Wu, Wd, gamma, beta):
    B, T, D = x.shape
    N = B * T
    E, _, F = Wg.shape
    capacity = math.ceil(N / E * 1.25)
    C = capacity
    x_flat = x.reshape(N, D)

    sel, gates, lb = pl.pallas_call(
        functools.partial(_routing_kernel, capacity=C),
        out_shape=[
            jax.ShapeDtypeStruct((N, E), jnp.int32),
            jax.ShapeDtypeStruct((N, E), jnp.float32),
            jax.ShapeDtypeStruct((1, 1), jnp.float32),
        ],
    )(x_flat, Wr)

    sel_t = sel.T.reshape(E, 1, N)
    gates_t = gates.T.reshape(E, 1, N)

    FB = 512
    NF = F // FB
    out = pl.pallas_call(
        functools.partial(_moe_kernel, nf=NF, n_experts=E, capacity=C),
        grid=(E, NF),
        in_specs=[
            pl.BlockSpec((N, D), lambda e, f: (0, 0)),
            pl.BlockSpec((1, 1, N), lambda e, f: (e, 0, 0)),
            pl.BlockSpec((1, 1, N), lambda e, f: (e, 0, 0)),
            pl.BlockSpec((1, D, FB), lambda e, f: (e, 0, f)),
            pl.BlockSpec((1, D, FB), lambda e, f: (e, 0, f)),
            pl.BlockSpec((1, FB, D), lambda e, f: (e, f, 0)),
            pl.BlockSpec((1, D), lambda e, f: (0, 0)),
            pl.BlockSpec((1, D), lambda e, f: (0, 0)),
        ],
        out_specs=pl.BlockSpec((N, D), lambda e, f: (0, 0)),
        out_shape=jax.ShapeDtypeStruct((N, D), jnp.float32),
        scratch_shapes=[
            pltpu.VMEM((C, D), jnp.bfloat16),
            pltpu.VMEM((C, N), jnp.bfloat16),
            pltpu.VMEM((C, D), jnp.float32),
            pltpu.VMEM((N, D), jnp.float32),
        ],
        compiler_params=pltpu.CompilerParams(
            dimension_semantics=("arbitrary", "arbitrary"),
        ),
    )(x_flat, sel_t, gates_t, Wg, Wu, Wd,
      gamma.reshape(1, D), beta.reshape(1, D))

    return out.reshape(B, T, D), lb[0, 0]
